# Initial kernel scaffold; baseline (speedup 1.0000x reference)
import jax
import jax.numpy as jnp
from jax.experimental import pallas as pl


def kernel(kp1, w_kp1, kp1_desc, desc2, homo12):
    raise NotImplementedError("write your pallas kernel here")


# speedup vs baseline: 13.9627x; 13.9627x over previous
import functools

import jax
import jax.numpy as jnp
from jax.experimental import pallas as pl
from jax.experimental.pallas import tpu as pltpu

GRID = 16.0
MARGIN = 1.0
NUM_NEG = 16
P = 128
Q = 256
BIGF = 3.0e38
BIGI = 2**30


def _chunk_ids(q):
    cid = jax.lax.broadcasted_iota(jnp.int32, (Q, 1), 0) + q * Q
    cx = (cid % 32).astype(jnp.float32) * GRID + GRID / 2.0
    cy = (cid // 32).astype(jnp.float32) * GRID + GRID / 2.0
    return cid, cx, cy


def _nearest4(x, y, nq):
    pm = x * x + y * y
    cand_v, cand_i = [], []
    for q in range(nq):
        cid, cx, cy = _chunk_ids(q)
        cm = cx * cx + cy * cy
        d2 = (pm + cm) - 2.0 * (cx * x + cy * y)
        dist = jnp.sqrt(jnp.maximum(d2, 1e-12))
        for _ in range(4):
            m = jnp.min(dist, axis=0, keepdims=True)
            idx = jnp.min(jnp.where(dist == m, cid, BIGI), axis=0,
                          keepdims=True)
            dist = jnp.where(cid == idx, BIGF, dist)
            cand_v.append(m)
            cand_i.append(idx)
    V = jnp.concatenate(cand_v, axis=0)
    I = jnp.concatenate(cand_i, axis=0)
    out = []
    for _ in range(4):
        m = jnp.min(V, axis=0, keepdims=True)
        idx = jnp.min(jnp.where(V == m, I, BIGI), axis=0, keepdims=True)
        V = jnp.where(I == idx, BIGF, V)
        out.append(idx)
    return out


def _loss_kernel(kpt_ref, wkpt_ref, descT_ref, draw_ref, homo_ref, out_ref,
                 *, nq):
    x = kpt_ref[0, 0:1, :]
    y = kpt_ref[0, 1:2, :]

    ids1 = _nearest4(x, y, nq)
    h = homo_ref[0]
    ids16 = []
    for idj in ids1:
        cx = (idj % 32).astype(jnp.float32) * GRID + GRID / 2.0
        cy = (idj // 32).astype(jnp.float32) * GRID + GRID / 2.0
        wz = h[2:3, 0:1] * cx + h[2:3, 1:2] * cy + h[2:3, 2:3] + 1e-8
        wx = (h[0:1, 0:1] * cx + h[0:1, 1:2] * cy + h[0:1, 2:3]) / wz
        wy = (h[1:2, 0:1] * cx + h[1:2, 1:2] * cy + h[1:2, 2:3]) / wz
        ids16.extend(_nearest4(wx, wy, nq))

    dT = descT_ref[0]
    n1 = dT / (jnp.sqrt(jnp.sum(dT * dT, axis=0, keepdims=True)) + 1e-8)

    sx = jnp.clip(wkpt_ref[0, 0:1, :] / GRID - 0.5, 0.0, 31.0)
    sy = jnp.clip(wkpt_ref[0, 1:2, :] / GRID - 0.5, 0.0, 31.0)
    x0 = jnp.clip(jnp.floor(sx), 0.0, 30.0)
    y0 = jnp.clip(jnp.floor(sy), 0.0, 30.0)
    fx = sx - x0
    fy = sy - y0
    m00 = y0.astype(jnp.int32) * 32 + x0.astype(jnp.int32)
    w00 = (1.0 - fx) * (1.0 - fy)
    w01 = fx * (1.0 - fy)
    w10 = (1.0 - fx) * fy
    w11 = fx * fy

    samp = jnp.zeros((draw_ref.shape[2], P), jnp.float32)
    neg_cands = []
    for q in range(nq):
        cid, _, _ = _chunk_ids(q)
        draw_q = draw_ref[0, q * Q:(q + 1) * Q, :]
        rn = jnp.sqrt(jnp.sum(draw_q * draw_q, axis=1, keepdims=True))
        n2q = draw_q / (rn + 1e-8)
        g = jax.lax.dot_general(n2q, n1, (((1,), (0,)), ((), ())),
                                preferred_element_type=jnp.float32)
        sim = 2.0 - 2.0 * g
        masked = (cid == ids16[0])
        for idj in ids16[1:]:
            masked = masked | (cid == idj)
        sim = jnp.where(masked, BIGF, sim)
        for _ in range(NUM_NEG):
            m = jnp.min(sim, axis=0, keepdims=True)
            idx = jnp.min(jnp.where(sim == m, cid, BIGI), axis=0,
                          keepdims=True)
            sim = jnp.where(cid == idx, BIGF, sim)
            neg_cands.append(m)

        s_q = (w00 * (cid == m00) + w01 * (cid == m00 + 1)
               + w10 * (cid == m00 + 32) + w11 * (cid == m00 + 33))
        samp = samp + jax.lax.dot_general(
            draw_q, s_q.astype(jnp.float32), (((0,), (0,)), ((), ())),
            preferred_element_type=jnp.float32)

    ns = jnp.sqrt(jnp.sum(samp * samp, axis=0, keepdims=True))
    nsamp = samp / (ns + 1e-8)
    pos = 2.0 - 2.0 * jnp.sum(n1 * nsamp, axis=0, keepdims=True)

    V = jnp.concatenate(neg_cands, axis=0)
    row = jax.lax.broadcasted_iota(jnp.int32, (V.shape[0], 1), 0)
    acc = jnp.zeros((1, P), jnp.float32)
    for _ in range(NUM_NEG):
        m = jnp.min(V, axis=0, keepdims=True)
        ridx = jnp.min(jnp.where(V == m, row, BIGI), axis=0, keepdims=True)
        V = jnp.where(row == ridx, BIGF, V)
        acc = acc + jnp.maximum(pos - m + MARGIN, 0.0)
    out_ref[0, 0] = acc


@jax.jit
def kernel(kp1, w_kp1, kp1_desc, desc2, homo12):
    b, n, c = kp1_desc.shape
    _, _, hh, ww = desc2.shape
    m = hh * ww
    nq = m // Q
    nb = n // P

    kpt = jnp.transpose(kp1, (0, 2, 1))
    wkpt = jnp.transpose(w_kp1, (0, 2, 1))
    desct = jnp.transpose(kp1_desc, (0, 2, 1))
    draw = jnp.transpose(desc2, (0, 2, 3, 1)).reshape(b, m, c)

    grid = (b, nb)
    out = pl.pallas_call(
        functools.partial(_loss_kernel, nq=nq),
        grid=grid,
        in_specs=[
            pl.BlockSpec((1, 2, P), lambda bi, ri: (bi, 0, ri)),
            pl.BlockSpec((1, 2, P), lambda bi, ri: (bi, 0, ri)),
            pl.BlockSpec((1, c, P), lambda bi, ri: (bi, 0, ri)),
            pl.BlockSpec((1, m, c), lambda bi, ri: (bi, 0, 0)),
            pl.BlockSpec((1, 3, 3), lambda bi, ri: (bi, 0, 0)),
        ],
        out_specs=pl.BlockSpec((1, 1, 1, P), lambda bi, ri: (bi, ri, 0, 0)),
        out_shape=jax.ShapeDtypeStruct((b, nb, 1, P), jnp.float32),
    )(kpt, wkpt, desct, draw, homo12)

    return jnp.sum(out) / (b * n * NUM_NEG)


# speedup vs baseline: 25.4328x; 1.8215x over previous
import functools

import jax
import jax.numpy as jnp
from jax.experimental import pallas as pl
from jax.experimental.pallas import tpu as pltpu

GRID = 16.0
MARGIN = 1.0
NUM_NEG = 16
P = 128
Q = 256
BIGF = 3.0e38
BIGI = 2**30


def _chunk_ids(q):
    cid = jax.lax.broadcasted_iota(jnp.int32, (Q, 1), 0) + q * Q
    cx = (cid % 32).astype(jnp.float32) * GRID + GRID / 2.0
    cy = (cid // 32).astype(jnp.float32) * GRID + GRID / 2.0
    return cid, cx, cy


def _top4_axis(p):
    ci = jax.lax.broadcasted_iota(jnp.int32, (32, 1), 0)
    cf = ci.astype(jnp.float32) * GRID + GRID / 2.0
    dd = (cf - p) * (cf - p)
    out = []
    for _ in range(4):
        m = jnp.min(dd, axis=0, keepdims=True)
        idx = jnp.min(jnp.where(dd == m, ci, BIGI), axis=0, keepdims=True)
        dd = jnp.where(ci == idx, BIGF, dd)
        out.append(idx)
    return out


def _nearest4(x, y, nq):
    del nq
    cols = _top4_axis(x)
    rows = _top4_axis(y)
    pm = x * x + y * y
    cand_v, cand_i = [], []
    for ri in rows:
        cyf = ri.astype(jnp.float32) * GRID + GRID / 2.0
        for cj in cols:
            cxf = cj.astype(jnp.float32) * GRID + GRID / 2.0
            cm = cxf * cxf + cyf * cyf
            d2 = (pm + cm) - 2.0 * (cxf * x + cyf * y)
            cand_v.append(jnp.sqrt(jnp.maximum(d2, 1e-12)))
            cand_i.append(ri * 32 + cj)
    V = jnp.concatenate(cand_v, axis=0)
    I = jnp.concatenate(cand_i, axis=0)
    out = []
    for _ in range(4):
        m = jnp.min(V, axis=0, keepdims=True)
        idx = jnp.min(jnp.where(V == m, I, BIGI), axis=0, keepdims=True)
        V = jnp.where(I == idx, BIGF, V)
        out.append(idx)
    return out


def _loss_kernel(kpt_ref, wkpt_ref, descT_ref, draw_ref, homo_ref, out_ref,
                 *, nq):
    x = kpt_ref[0, 0:1, :]
    y = kpt_ref[0, 1:2, :]

    ids1 = _nearest4(x, y, nq)
    h = homo_ref[0]
    ids16 = []
    for idj in ids1:
        cx = (idj % 32).astype(jnp.float32) * GRID + GRID / 2.0
        cy = (idj // 32).astype(jnp.float32) * GRID + GRID / 2.0
        wz = h[2:3, 0:1] * cx + h[2:3, 1:2] * cy + h[2:3, 2:3] + 1e-8
        wx = (h[0:1, 0:1] * cx + h[0:1, 1:2] * cy + h[0:1, 2:3]) / wz
        wy = (h[1:2, 0:1] * cx + h[1:2, 1:2] * cy + h[1:2, 2:3]) / wz
        ids16.extend(_nearest4(wx, wy, nq))

    dT = descT_ref[0]
    n1 = dT / (jnp.sqrt(jnp.sum(dT * dT, axis=0, keepdims=True)) + 1e-8)

    sx = jnp.clip(wkpt_ref[0, 0:1, :] / GRID - 0.5, 0.0, 31.0)
    sy = jnp.clip(wkpt_ref[0, 1:2, :] / GRID - 0.5, 0.0, 31.0)
    x0 = jnp.clip(jnp.floor(sx), 0.0, 30.0)
    y0 = jnp.clip(jnp.floor(sy), 0.0, 30.0)
    fx = sx - x0
    fy = sy - y0
    m00 = y0.astype(jnp.int32) * 32 + x0.astype(jnp.int32)
    w00 = (1.0 - fx) * (1.0 - fy)
    w01 = fx * (1.0 - fy)
    w10 = (1.0 - fx) * fy
    w11 = fx * fy

    samp = jnp.zeros((draw_ref.shape[2], P), jnp.float32)
    neg_cands = []
    for q in range(nq):
        cid, _, _ = _chunk_ids(q)
        draw_q = draw_ref[0, q * Q:(q + 1) * Q, :]
        rn = jnp.sqrt(jnp.sum(draw_q * draw_q, axis=1, keepdims=True))
        n2q = draw_q / (rn + 1e-8)
        g = jax.lax.dot_general(n2q, n1, (((1,), (0,)), ((), ())),
                                preferred_element_type=jnp.float32)
        sim = 2.0 - 2.0 * g
        masked = (cid == ids16[0])
        for idj in ids16[1:]:
            masked = masked | (cid == idj)
        sim = jnp.where(masked, BIGF, sim)
        for _ in range(NUM_NEG):
            m = jnp.min(sim, axis=0, keepdims=True)
            idx = jnp.min(jnp.where(sim == m, cid, BIGI), axis=0,
                          keepdims=True)
            sim = jnp.where(cid == idx, BIGF, sim)
            neg_cands.append(m)

        s_q = (w00 * (cid == m00) + w01 * (cid == m00 + 1)
               + w10 * (cid == m00 + 32) + w11 * (cid == m00 + 33))
        samp = samp + jax.lax.dot_general(
            draw_q, s_q.astype(jnp.float32), (((0,), (0,)), ((), ())),
            preferred_element_type=jnp.float32)

    ns = jnp.sqrt(jnp.sum(samp * samp, axis=0, keepdims=True))
    nsamp = samp / (ns + 1e-8)
    pos = 2.0 - 2.0 * jnp.sum(n1 * nsamp, axis=0, keepdims=True)

    V = jnp.concatenate(neg_cands, axis=0)
    row = jax.lax.broadcasted_iota(jnp.int32, (V.shape[0], 1), 0)
    acc = jnp.zeros((1, P), jnp.float32)
    for _ in range(NUM_NEG):
        m = jnp.min(V, axis=0, keepdims=True)
        ridx = jnp.min(jnp.where(V == m, row, BIGI), axis=0, keepdims=True)
        V = jnp.where(row == ridx, BIGF, V)
        acc = acc + jnp.maximum(pos - m + MARGIN, 0.0)
    out_ref[0, 0] = acc


@jax.jit
def kernel(kp1, w_kp1, kp1_desc, desc2, homo12):
    b, n, c = kp1_desc.shape
    _, _, hh, ww = desc2.shape
    m = hh * ww
    nq = m // Q
    nb = n // P

    kpt = jnp.transpose(kp1, (0, 2, 1))
    wkpt = jnp.transpose(w_kp1, (0, 2, 1))
    desct = jnp.transpose(kp1_desc, (0, 2, 1))
    draw = jnp.transpose(desc2, (0, 2, 3, 1)).reshape(b, m, c)

    grid = (b, nb)
    out = pl.pallas_call(
        functools.partial(_loss_kernel, nq=nq),
        grid=grid,
        in_specs=[
            pl.BlockSpec((1, 2, P), lambda bi, ri: (bi, 0, ri)),
            pl.BlockSpec((1, 2, P), lambda bi, ri: (bi, 0, ri)),
            pl.BlockSpec((1, c, P), lambda bi, ri: (bi, 0, ri)),
            pl.BlockSpec((1, m, c), lambda bi, ri: (bi, 0, 0)),
            pl.BlockSpec((1, 3, 3), lambda bi, ri: (bi, 0, 0)),
        ],
        out_specs=pl.BlockSpec((1, 1, 1, P), lambda bi, ri: (bi, ri, 0, 0)),
        out_shape=jax.ShapeDtypeStruct((b, nb, 1, P), jnp.float32),
    )(kpt, wkpt, desct, draw, homo12)

    return jnp.sum(out) / (b * n * NUM_NEG)


# speedup vs baseline: 30.5062x; 1.1995x over previous
import functools

import jax
import jax.numpy as jnp
from jax.experimental import pallas as pl
from jax.experimental.pallas import tpu as pltpu

GRID = 16.0
MARGIN = 1.0
NUM_NEG = 16
P = 128
Q = 256
BIGF = 3.0e38
BIGI = 2**30


def _chunk_ids(q):
    cid = jax.lax.broadcasted_iota(jnp.int32, (Q, 1), 0) + q * Q
    cx = (cid % 32).astype(jnp.float32) * GRID + GRID / 2.0
    cy = (cid // 32).astype(jnp.float32) * GRID + GRID / 2.0
    return cid, cx, cy


def _top4_axis(p):
    ci = jax.lax.broadcasted_iota(jnp.int32, (32, 1), 0)
    cf = ci.astype(jnp.float32) * GRID + GRID / 2.0
    dd = (cf - p) * (cf - p)
    out = []
    for _ in range(4):
        m = jnp.min(dd, axis=0, keepdims=True)
        idx = jnp.min(jnp.where(dd == m, ci, BIGI), axis=0, keepdims=True)
        dd = jnp.where(ci == idx, BIGF, dd)
        out.append(idx)
    return out


def _nearest4(x, y, nq):
    del nq
    cols = _top4_axis(x)
    rows = _top4_axis(y)
    pm = x * x + y * y
    cand_v, cand_i = [], []
    for ri in rows:
        cyf = ri.astype(jnp.float32) * GRID + GRID / 2.0
        for cj in cols:
            cxf = cj.astype(jnp.float32) * GRID + GRID / 2.0
            cm = cxf * cxf + cyf * cyf
            d2 = (pm + cm) - 2.0 * (cxf * x + cyf * y)
            cand_v.append(jnp.sqrt(jnp.maximum(d2, 1e-12)))
            cand_i.append(ri * 32 + cj)
    V = jnp.concatenate(cand_v, axis=0)
    I = jnp.concatenate(cand_i, axis=0)
    out = []
    for _ in range(4):
        m = jnp.min(V, axis=0, keepdims=True)
        idx = jnp.min(jnp.where(V == m, I, BIGI), axis=0, keepdims=True)
        V = jnp.where(I == idx, BIGF, V)
        out.append(idx)
    return out


def _norm_rows_kernel(draw_ref, n2_ref):
    d = draw_ref[0]
    n2_ref[0] = d / (jnp.sqrt(jnp.sum(d * d, axis=1, keepdims=True)) + 1e-8)


def _norm_cols_kernel(descT_ref, n1_ref):
    t = descT_ref[0]
    n1_ref[0] = t / (jnp.sqrt(jnp.sum(t * t, axis=0, keepdims=True)) + 1e-8)


def _loss_kernel(kpt_ref, wkpt_ref, descT_ref, draw_ref, n2_ref, homo_ref,
                 out_ref, *, nq):
    x = kpt_ref[0, 0:1, :]
    y = kpt_ref[0, 1:2, :]

    ids1 = _nearest4(x, y, nq)
    h = homo_ref[0]
    ids16 = []
    for idj in ids1:
        cx = (idj % 32).astype(jnp.float32) * GRID + GRID / 2.0
        cy = (idj // 32).astype(jnp.float32) * GRID + GRID / 2.0
        wz = h[2:3, 0:1] * cx + h[2:3, 1:2] * cy + h[2:3, 2:3] + 1e-8
        wx = (h[0:1, 0:1] * cx + h[0:1, 1:2] * cy + h[0:1, 2:3]) / wz
        wy = (h[1:2, 0:1] * cx + h[1:2, 1:2] * cy + h[1:2, 2:3]) / wz
        ids16.extend(_nearest4(wx, wy, nq))

    n1 = descT_ref[0]

    sx = jnp.clip(wkpt_ref[0, 0:1, :] / GRID - 0.5, 0.0, 31.0)
    sy = jnp.clip(wkpt_ref[0, 1:2, :] / GRID - 0.5, 0.0, 31.0)
    x0 = jnp.clip(jnp.floor(sx), 0.0, 30.0)
    y0 = jnp.clip(jnp.floor(sy), 0.0, 30.0)
    fx = sx - x0
    fy = sy - y0
    m00 = y0.astype(jnp.int32) * 32 + x0.astype(jnp.int32)
    w00 = (1.0 - fx) * (1.0 - fy)
    w01 = fx * (1.0 - fy)
    w10 = (1.0 - fx) * fy
    w11 = fx * fy

    samp = jnp.zeros((draw_ref.shape[2], P), jnp.float32)
    neg_cands = []
    for q in range(nq):
        cid, _, _ = _chunk_ids(q)
        draw_q = draw_ref[0, q * Q:(q + 1) * Q, :]
        n2q = n2_ref[0, q * Q:(q + 1) * Q, :]
        g = jax.lax.dot_general(n2q, n1, (((1,), (0,)), ((), ())),
                                preferred_element_type=jnp.float32)
        sim = 2.0 - 2.0 * g
        masked = (cid == ids16[0])
        for idj in ids16[1:]:
            masked = masked | (cid == idj)
        sim = jnp.where(masked, BIGF, sim)
        for _ in range(NUM_NEG):
            m = jnp.min(sim, axis=0, keepdims=True)
            sim = jnp.where(sim == m, BIGF, sim)
            neg_cands.append(m)

        s_q = (w00 * (cid == m00) + w01 * (cid == m00 + 1)
               + w10 * (cid == m00 + 32) + w11 * (cid == m00 + 33))
        samp = samp + jax.lax.dot_general(
            draw_q, s_q.astype(jnp.float32), (((0,), (0,)), ((), ())),
            preferred_element_type=jnp.float32)

    ns = jnp.sqrt(jnp.sum(samp * samp, axis=0, keepdims=True))
    nsamp = samp / (ns + 1e-8)
    pos = 2.0 - 2.0 * jnp.sum(n1 * nsamp, axis=0, keepdims=True)

    V = jnp.concatenate(neg_cands, axis=0)
    acc = jnp.zeros((1, P), jnp.float32)
    for _ in range(NUM_NEG):
        m = jnp.min(V, axis=0, keepdims=True)
        V = jnp.where(V == m, BIGF, V)
        acc = acc + jnp.maximum(pos - m + MARGIN, 0.0)
    out_ref[0, 0] = acc


@jax.jit
def kernel(kp1, w_kp1, kp1_desc, desc2, homo12):
    b, n, c = kp1_desc.shape
    _, _, hh, ww = desc2.shape
    m = hh * ww
    nq = m // Q
    nb = n // P

    kpt = jnp.transpose(kp1, (0, 2, 1))
    wkpt = jnp.transpose(w_kp1, (0, 2, 1))
    desct = jnp.transpose(kp1_desc, (0, 2, 1))
    draw = jnp.transpose(desc2, (0, 2, 3, 1)).reshape(b, m, c)

    n2 = pl.pallas_call(
        _norm_rows_kernel,
        grid=(b, nq),
        in_specs=[pl.BlockSpec((1, Q, c), lambda bi, qi: (bi, qi, 0))],
        out_specs=pl.BlockSpec((1, Q, c), lambda bi, qi: (bi, qi, 0)),
        out_shape=jax.ShapeDtypeStruct((b, m, c), jnp.float32),
    )(draw)

    n1 = pl.pallas_call(
        _norm_cols_kernel,
        grid=(b, nb),
        in_specs=[pl.BlockSpec((1, c, P), lambda bi, ri: (bi, 0, ri))],
        out_specs=pl.BlockSpec((1, c, P), lambda bi, ri: (bi, 0, ri)),
        out_shape=jax.ShapeDtypeStruct((b, c, n), jnp.float32),
    )(desct)

    grid = (b, nb)
    out = pl.pallas_call(
        functools.partial(_loss_kernel, nq=nq),
        grid=grid,
        in_specs=[
            pl.BlockSpec((1, 2, P), lambda bi, ri: (bi, 0, ri)),
            pl.BlockSpec((1, 2, P), lambda bi, ri: (bi, 0, ri)),
            pl.BlockSpec((1, c, P), lambda bi, ri: (bi, 0, ri)),
            pl.BlockSpec((1, m, c), lambda bi, ri: (bi, 0, 0)),
            pl.BlockSpec((1, m, c), lambda bi, ri: (bi, 0, 0)),
            pl.BlockSpec((1, 3, 3), lambda bi, ri: (bi, 0, 0)),
        ],
        out_specs=pl.BlockSpec((1, 1, 1, P), lambda bi, ri: (bi, ri, 0, 0)),
        out_shape=jax.ShapeDtypeStruct((b, nb, 1, P), jnp.float32),
    )(kpt, wkpt, n1, draw, n2, homo12)

    return jnp.sum(out) / (b * n * NUM_NEG)


# speedup vs baseline: 38.3448x; 1.2570x over previous
import functools

import jax
import jax.numpy as jnp
from jax.experimental import pallas as pl
from jax.experimental.pallas import tpu as pltpu

GRID = 16.0
MARGIN = 1.0
NUM_NEG = 16
P = 128
Q = 256
BIGF = 3.0e38
BIGI = 2**30


def _chunk_ids(q):
    cid = jax.lax.broadcasted_iota(jnp.int32, (Q, 1), 0) + q * Q
    cx = (cid % 32).astype(jnp.float32) * GRID + GRID / 2.0
    cy = (cid // 32).astype(jnp.float32) * GRID + GRID / 2.0
    return cid, cx, cy


def _top4_axis(p):
    ci = jax.lax.broadcasted_iota(jnp.int32, (32, 1), 0)
    cf = ci.astype(jnp.float32) * GRID + GRID / 2.0
    dd = (cf - p) * (cf - p)
    out = []
    for _ in range(4):
        m = jnp.min(dd, axis=0, keepdims=True)
        idx = jnp.min(jnp.where(dd == m, ci, BIGI), axis=0, keepdims=True)
        dd = jnp.where(ci == idx, BIGF, dd)
        out.append(idx)
    return out


def _nearest4(x, y, nq):
    del nq
    cols = _top4_axis(x)
    rows = _top4_axis(y)
    pm = x * x + y * y
    cand_v, cand_i = [], []
    for ri in rows:
        cyf = ri.astype(jnp.float32) * GRID + GRID / 2.0
        for cj in cols:
            cxf = cj.astype(jnp.float32) * GRID + GRID / 2.0
            cm = cxf * cxf + cyf * cyf
            d2 = (pm + cm) - 2.0 * (cxf * x + cyf * y)
            cand_v.append(jnp.sqrt(jnp.maximum(d2, 1e-12)))
            cand_i.append(ri * 32 + cj)
    V = jnp.concatenate(cand_v, axis=0)
    I = jnp.concatenate(cand_i, axis=0)
    out = []
    for _ in range(4):
        m = jnp.min(V, axis=0, keepdims=True)
        idx = jnp.min(jnp.where(V == m, I, BIGI), axis=0, keepdims=True)
        V = jnp.where(I == idx, BIGF, V)
        out.append(idx)
    return out


def _loss_kernel(kpt_ref, wkpt_ref, desc_ref, d2r_ref, homo_ref,
                 out_ref, n2_scr, *, nq):
    @pl.when(pl.program_id(1) == 0)
    def _build_n2():
        for q in range(nq):
            d = d2r_ref[0, :, q * Q:(q + 1) * Q]
            rn = jnp.sqrt(jnp.sum(d * d, axis=0, keepdims=True))
            n2_scr[:, q * Q:(q + 1) * Q] = d / (rn + 1e-8)

    x = kpt_ref[0, 0:1, :]
    y = kpt_ref[0, 1:2, :]

    ids1 = _nearest4(x, y, nq)
    h = homo_ref[0]
    ids16 = []
    for idj in ids1:
        cx = (idj % 32).astype(jnp.float32) * GRID + GRID / 2.0
        cy = (idj // 32).astype(jnp.float32) * GRID + GRID / 2.0
        wz = h[2:3, 0:1] * cx + h[2:3, 1:2] * cy + h[2:3, 2:3] + 1e-8
        wx = (h[0:1, 0:1] * cx + h[0:1, 1:2] * cy + h[0:1, 2:3]) / wz
        wy = (h[1:2, 0:1] * cx + h[1:2, 1:2] * cy + h[1:2, 2:3]) / wz
        ids16.extend(_nearest4(wx, wy, nq))

    dpc = desc_ref[0]
    n1pc = dpc / (jnp.sqrt(jnp.sum(dpc * dpc, axis=1, keepdims=True)) + 1e-8)

    sx = jnp.clip(wkpt_ref[0, 0:1, :] / GRID - 0.5, 0.0, 31.0)
    sy = jnp.clip(wkpt_ref[0, 1:2, :] / GRID - 0.5, 0.0, 31.0)
    x0 = jnp.clip(jnp.floor(sx), 0.0, 30.0)
    y0 = jnp.clip(jnp.floor(sy), 0.0, 30.0)
    fx = sx - x0
    fy = sy - y0
    m00 = y0.astype(jnp.int32) * 32 + x0.astype(jnp.int32)
    w00 = (1.0 - fx) * (1.0 - fy)
    w01 = fx * (1.0 - fy)
    w10 = (1.0 - fx) * fy
    w11 = fx * fy

    samp = jnp.zeros((P, d2r_ref.shape[1]), jnp.float32)
    neg_cands = []
    for q in range(nq):
        cid, _, _ = _chunk_ids(q)
        n2q = n2_scr[:, q * Q:(q + 1) * Q]
        g = jax.lax.dot_general(n2q, n1pc, (((0,), (1,)), ((), ())),
                                preferred_element_type=jnp.float32)
        sim = 2.0 - 2.0 * g
        masked = (cid == ids16[0])
        for idj in ids16[1:]:
            masked = masked | (cid == idj)
        sim = jnp.where(masked, BIGF, sim)
        for _ in range(NUM_NEG):
            m = jnp.min(sim, axis=0, keepdims=True)
            sim = jnp.where(sim == m, BIGF, sim)
            neg_cands.append(m)

        s_q = (w00 * (cid == m00) + w01 * (cid == m00 + 1)
               + w10 * (cid == m00 + 32) + w11 * (cid == m00 + 33))
        samp = samp + jax.lax.dot_general(
            s_q.astype(jnp.float32), d2r_ref[0, :, q * Q:(q + 1) * Q],
            (((0,), (1,)), ((), ())),
            preferred_element_type=jnp.float32)

    ns = jnp.sqrt(jnp.sum(samp * samp, axis=1, keepdims=True))
    nsamp = samp / (ns + 1e-8)
    posc = 2.0 - 2.0 * jnp.sum(n1pc * nsamp, axis=1, keepdims=True)
    pos = jnp.transpose(posc, (1, 0))

    V = jnp.concatenate(neg_cands, axis=0)
    acc = jnp.zeros((1, P), jnp.float32)
    for _ in range(NUM_NEG):
        m = jnp.min(V, axis=0, keepdims=True)
        V = jnp.where(V == m, BIGF, V)
        acc = acc + jnp.maximum(pos - m + MARGIN, 0.0)
    out_ref[0, 0] = acc


@jax.jit
def kernel(kp1, w_kp1, kp1_desc, desc2, homo12):
    b, n, c = kp1_desc.shape
    _, _, hh, ww = desc2.shape
    m = hh * ww
    nq = m // Q
    nb = n // P

    kpt = jnp.transpose(kp1, (0, 2, 1))
    wkpt = jnp.transpose(w_kp1, (0, 2, 1))
    d2r = desc2.reshape(b, c, m)

    grid = (b, nb)
    out = pl.pallas_call(
        functools.partial(_loss_kernel, nq=nq),
        grid=grid,
        in_specs=[
            pl.BlockSpec((1, 2, P), lambda bi, ri: (bi, 0, ri)),
            pl.BlockSpec((1, 2, P), lambda bi, ri: (bi, 0, ri)),
            pl.BlockSpec((1, P, c), lambda bi, ri: (bi, ri, 0)),
            pl.BlockSpec((1, c, m), lambda bi, ri: (bi, 0, 0)),
            pl.BlockSpec((1, 3, 3), lambda bi, ri: (bi, 0, 0)),
        ],
        out_specs=pl.BlockSpec((1, 1, 1, P), lambda bi, ri: (bi, ri, 0, 0)),
        out_shape=jax.ShapeDtypeStruct((b, nb, 1, P), jnp.float32),
        scratch_shapes=[pltpu.VMEM((c, m), jnp.float32)],
    )(kpt, wkpt, kp1_desc, d2r, homo12)

    return jnp.sum(out) / (b * n * NUM_NEG)


# speedup vs baseline: 39.7001x; 1.0353x over previous
import functools

import jax
import jax.numpy as jnp
from jax.experimental import pallas as pl
from jax.experimental.pallas import tpu as pltpu

GRID = 16.0
MARGIN = 1.0
NUM_NEG = 16
P = 128
Q = 256
BIGF = 3.0e38
BIGI = 2**30


def _chunk_ids(q):
    cid = jax.lax.broadcasted_iota(jnp.int32, (Q, 1), 0) + q * Q
    cx = (cid % 32).astype(jnp.float32) * GRID + GRID / 2.0
    cy = (cid // 32).astype(jnp.float32) * GRID + GRID / 2.0
    return cid, cx, cy


def _top4_axis(p):
    ci = jax.lax.broadcasted_iota(jnp.int32, (32, 1), 0)
    cf = ci.astype(jnp.float32) * GRID + GRID / 2.0
    dd = (cf - p) * (cf - p)
    out = []
    for _ in range(4):
        m = jnp.min(dd, axis=0, keepdims=True)
        idx = jnp.min(jnp.where(dd == m, ci, BIGI), axis=0, keepdims=True)
        dd = jnp.where(ci == idx, BIGF, dd)
        out.append(idx)
    return out


def _nearest4(x, y, nq):
    del nq
    cols = _top4_axis(x)
    rows = _top4_axis(y)
    pm = x * x + y * y
    cand_v, cand_i = [], []
    for ri in rows:
        cyf = ri.astype(jnp.float32) * GRID + GRID / 2.0
        for cj in cols:
            cxf = cj.astype(jnp.float32) * GRID + GRID / 2.0
            cm = cxf * cxf + cyf * cyf
            d2 = (pm + cm) - 2.0 * (cxf * x + cyf * y)
            cand_v.append(jnp.sqrt(jnp.maximum(d2, 1e-12)))
            cand_i.append(ri * 32 + cj)
    V = jnp.concatenate(cand_v, axis=0)
    I = jnp.concatenate(cand_i, axis=0)
    out = []
    for _ in range(4):
        m = jnp.min(V, axis=0, keepdims=True)
        idx = jnp.min(jnp.where(V == m, I, BIGI), axis=0, keepdims=True)
        V = jnp.where(I == idx, BIGF, V)
        out.append(idx)
    return out


def _loss_kernel(kpt_ref, wkpt_ref, desc_ref, d2r_ref, homo_ref,
                 out_ref, n2_scr, *, nq):
    @pl.when(pl.program_id(1) == 0)
    def _build_n2():
        for q in range(nq):
            d = d2r_ref[0, :, q * Q:(q + 1) * Q]
            rn = jnp.sqrt(jnp.sum(d * d, axis=0, keepdims=True))
            n2_scr[:, q * Q:(q + 1) * Q] = d / (rn + 1e-8)

    x = kpt_ref[0, 0:1, :]
    y = kpt_ref[0, 1:2, :]

    ids1 = _nearest4(x, y, nq)
    h = homo_ref[0]
    ids16 = []
    for idj in ids1:
        cx = (idj % 32).astype(jnp.float32) * GRID + GRID / 2.0
        cy = (idj // 32).astype(jnp.float32) * GRID + GRID / 2.0
        wz = h[2:3, 0:1] * cx + h[2:3, 1:2] * cy + h[2:3, 2:3] + 1e-8
        wx = (h[0:1, 0:1] * cx + h[0:1, 1:2] * cy + h[0:1, 2:3]) / wz
        wy = (h[1:2, 0:1] * cx + h[1:2, 1:2] * cy + h[1:2, 2:3]) / wz
        ids16.extend(_nearest4(wx, wy, nq))

    dpc = desc_ref[0]
    n1pc = dpc / (jnp.sqrt(jnp.sum(dpc * dpc, axis=1, keepdims=True)) + 1e-8)

    sx = jnp.clip(wkpt_ref[0, 0:1, :] / GRID - 0.5, 0.0, 31.0)
    sy = jnp.clip(wkpt_ref[0, 1:2, :] / GRID - 0.5, 0.0, 31.0)
    x0 = jnp.clip(jnp.floor(sx), 0.0, 30.0)
    y0 = jnp.clip(jnp.floor(sy), 0.0, 30.0)
    fx = sx - x0
    fy = sy - y0
    m00 = y0.astype(jnp.int32) * 32 + x0.astype(jnp.int32)
    w00 = (1.0 - fx) * (1.0 - fy)
    w01 = fx * (1.0 - fy)
    w10 = (1.0 - fx) * fy
    w11 = fx * fy

    samp = jnp.zeros((P, d2r_ref.shape[1]), jnp.float32)
    l0, l1, l2, l3 = [], [], [], []
    for q in range(nq):
        cid, _, _ = _chunk_ids(q)
        n2q = n2_scr[:, q * Q:(q + 1) * Q]
        g = jax.lax.dot_general(n2q, n1pc, (((0,), (1,)), ((), ())),
                                preferred_element_type=jnp.float32)
        sim = 2.0 - 2.0 * g
        masked = (cid == ids16[0])
        for idj in ids16[1:]:
            masked = masked | (cid == idj)
        sim = jnp.where(masked, BIGF, sim)
        s0, s1 = sim[:Q // 4], sim[Q // 4:Q // 2]
        s2, s3 = sim[Q // 2:3 * Q // 4], sim[3 * Q // 4:]
        a0, a1 = jnp.minimum(s0, s1), jnp.maximum(s0, s1)
        b0, b1 = jnp.minimum(s2, s3), jnp.maximum(s2, s3)
        c0, c2 = jnp.minimum(a0, b0), jnp.maximum(a0, b0)
        c1, c3 = jnp.minimum(a1, b1), jnp.maximum(a1, b1)
        d1, d2 = jnp.minimum(c2, c1), jnp.maximum(c2, c1)
        l0.append(c0)
        l1.append(d1)
        l2.append(d2)
        l3.append(c3)

        s_q = (w00 * (cid == m00) + w01 * (cid == m00 + 1)
               + w10 * (cid == m00 + 32) + w11 * (cid == m00 + 33))
        samp = samp + jax.lax.dot_general(
            s_q.astype(jnp.float32), d2r_ref[0, :, q * Q:(q + 1) * Q],
            (((0,), (1,)), ((), ())),
            preferred_element_type=jnp.float32)

    ns = jnp.sqrt(jnp.sum(samp * samp, axis=1, keepdims=True))
    nsamp = samp / (ns + 1e-8)
    posc = 2.0 - 2.0 * jnp.sum(n1pc * nsamp, axis=1, keepdims=True)
    pos = jnp.transpose(posc, (1, 0))

    A = jnp.concatenate(l0, axis=0)
    S2 = jnp.concatenate(l1, axis=0)
    S3 = jnp.concatenate(l2, axis=0)
    S4 = jnp.concatenate(l3, axis=0)
    acc = jnp.zeros((1, P), jnp.float32)
    for _ in range(NUM_NEG):
        m = jnp.min(A, axis=0, keepdims=True)
        eq = A == m
        A = jnp.where(eq, S2, A)
        S2 = jnp.where(eq, S3, S2)
        S3 = jnp.where(eq, S4, S3)
        S4 = jnp.where(eq, BIGF, S4)
        acc = acc + jnp.maximum(pos - m + MARGIN, 0.0)
    out_ref[0, 0] = acc


@jax.jit
def kernel(kp1, w_kp1, kp1_desc, desc2, homo12):
    b, n, c = kp1_desc.shape
    _, _, hh, ww = desc2.shape
    m = hh * ww
    nq = m // Q
    nb = n // P

    kpt = jnp.transpose(kp1, (0, 2, 1))
    wkpt = jnp.transpose(w_kp1, (0, 2, 1))
    d2r = desc2.reshape(b, c, m)

    grid = (b, nb)
    out = pl.pallas_call(
        functools.partial(_loss_kernel, nq=nq),
        grid=grid,
        in_specs=[
            pl.BlockSpec((1, 2, P), lambda bi, ri: (bi, 0, ri)),
            pl.BlockSpec((1, 2, P), lambda bi, ri: (bi, 0, ri)),
            pl.BlockSpec((1, P, c), lambda bi, ri: (bi, ri, 0)),
            pl.BlockSpec((1, c, m), lambda bi, ri: (bi, 0, 0)),
            pl.BlockSpec((1, 3, 3), lambda bi, ri: (bi, 0, 0)),
        ],
        out_specs=pl.BlockSpec((1, 1, 1, P), lambda bi, ri: (bi, ri, 0, 0)),
        out_shape=jax.ShapeDtypeStruct((b, nb, 1, P), jnp.float32),
        scratch_shapes=[pltpu.VMEM((c, m), jnp.float32)],
    )(kpt, wkpt, kp1_desc, d2r, homo12)

    return jnp.sum(out) / (b * n * NUM_NEG)
